# Initial kernel scaffold; baseline (speedup 1.0000x reference)
import jax
import jax.numpy as jnp
from jax.experimental import pallas as pl


def kernel(src_seq, H, adj, W1, b1, W2, b2, Wm, bm, Wlin, blin, Wih0, Whh0, bih0, bhh0, Wih1, Whh1, bih1, bhh1, Wq, bq, Wk, bk, Wv, bv, Wo, bo, ln_g, ln_b, Wl, bl, Wp, n_hid):
    raise NotImplementedError("write your pallas kernel here")


# speedup vs baseline: 1.2536x; 1.2536x over previous
import math

import jax
import jax.numpy as jnp
from jax.experimental import pallas as pl
from jax.experimental.pallas import tpu as pltpu

B, S, T, F = 8, 1024, 16, 64
NB = B * S
RNN = 64
NHID = 256
NH, DK, DV = 4, 16, 16
R_GRU = 256
R_ATT = 256
F32 = jnp.float32
BF16 = jnp.bfloat16


def _mm(a, b):
    return jax.lax.dot_general(a.astype(BF16), b, (((a.ndim - 1,), (0,)),
                                                   ((), ())),
                               preferred_element_type=F32)


def _hn_kernel(h_ref, wmt_ref, bm_ref, out_ref):
    out_ref[...] = (_mm(h_ref[...], wmt_ref[...]) + bm_ref[...]).astype(BF16)


def _hn_call(H, WmT, bm2):
    return pl.pallas_call(
        _hn_kernel,
        out_shape=jax.ShapeDtypeStruct((S, S), BF16),
    )(H, WmT, bm2)


def _gcn_kernel(x_ref, adj_ref, hn_ref, w1_ref, b1_ref, w2_ref, b2t_ref,
                wlt_ref, blin_ref, out_ref):
    xb = x_ref[0]
    adj = adj_ref[...]
    hn = hn_ref[...]
    w1 = w1_ref[...]
    w2 = w2_ref[...]
    b1 = b1_ref[...]
    s1_parts = []
    s2_parts = []
    for tt in range(8):
        xt = xb[:, tt * 64:(tt + 1) * 64]
        t1 = _mm(xt, w1).astype(BF16)
        o1 = jax.nn.relu(_mm(adj, t1) + b1)
        o2 = jax.nn.relu(_mm(hn, t1) + b1)
        s1_parts.append(_mm(o1, w2).astype(BF16))
        s2_parts.append(_mm(o2, w2).astype(BF16))
    s1 = jnp.concatenate(s1_parts, axis=1)
    s2 = jnp.concatenate(s2_parts, axis=1)
    y = _mm(adj, s1) + _mm(hn, s2) + b2t_ref[...]
    wlt = wlt_ref[...]
    blin = blin_ref[...]
    for tt in range(8):
        sl = slice(tt * 64, (tt + 1) * 64)
        out_ref[tt, 0] = (_mm(y[:, sl], wlt) + blin).astype(BF16)


def _gcn_call(xr, adj_bf, hn_bf, W1b, b12, W2b, b2t, WlinTb, blin2):
    cb = lambda r, cc: pl.BlockSpec((r, cc), lambda b, h: (0, 0))
    return pl.pallas_call(
        _gcn_kernel,
        grid=(B, 2),
        in_specs=[
            pl.BlockSpec((1, S, 512), lambda b, h: (b, 0, h)),
            cb(S, S), cb(S, S),
            cb(64, NHID), cb(1, NHID),
            cb(NHID, 64), cb(1, 512),
            cb(64, 64), cb(1, 64),
        ],
        out_specs=pl.BlockSpec((8, 1, S, 64), lambda b, h: (h, b, 0, 0)),
        out_shape=jax.ShapeDtypeStruct((T, B, S, 64), BF16),
    )(xr, adj_bf, hn_bf, W1b, b12, W2b, b2t, WlinTb, blin2)


def _gru_kernel(x_ref, wih0t_ref, bih0_ref, whhblk_ref, bhh0_ref, bhh1_ref,
                wih1t_ref, bih1_ref, out_ref):
    x3 = x_ref[...]
    x2 = x3.reshape(T * R_GRU, 64)
    gx0 = _mm(x2, wih0t_ref[...]) + bih0_ref[...]
    gx0 = gx0.reshape(T, R_GRU, 3 * RNN)
    whhblk = whhblk_ref[...]
    wih1t = wih1t_ref[...]
    bhh0 = bhh0_ref[...]
    bhh1 = bhh1_ref[...]
    bih1 = bih1_ref[...]
    h0 = jnp.zeros((R_GRU, RNN), F32)
    h1 = jnp.zeros((R_GRU, RNN), F32)
    for t in range(T):
        hcat = jnp.concatenate([h0, h1], axis=1)
        gh = _mm(hcat, whhblk)
        gh0 = gh[:, :192] + bhh0
        gh1 = gh[:, 192:] + bhh1
        g0 = gx0[t]
        r0 = jax.nn.sigmoid(g0[:, :64] + gh0[:, :64])
        z0 = jax.nn.sigmoid(g0[:, 64:128] + gh0[:, 64:128])
        n0 = jnp.tanh(g0[:, 128:] + r0 * gh0[:, 128:])
        h0 = (1.0 - z0) * n0 + z0 * h0
        g1 = _mm(h0, wih1t) + bih1
        r1 = jax.nn.sigmoid(g1[:, :64] + gh1[:, :64])
        z1 = jax.nn.sigmoid(g1[:, 64:128] + gh1[:, 64:128])
        n1 = jnp.tanh(g1[:, 128:] + r1 * gh1[:, 128:])
        h1 = (1.0 - z1) * n1 + z1 * h1
        out_ref[t] = h1


def _gru_call(s_tm, Wih0Tb, bih0, WhhBlkb, bhh0, bhh1, Wih1Tb, bih1):
    n_prog = NB // R_GRU
    c = lambda r, cc: pl.BlockSpec((r, cc), lambda i: (0, 0))
    return pl.pallas_call(
        _gru_kernel,
        grid=(n_prog,),
        in_specs=[
            pl.BlockSpec((T, R_GRU, 64), lambda r: (0, r, 0)),
            c(64, 192), c(1, 192),
            c(128, 384), c(1, 192), c(1, 192),
            c(64, 192), c(1, 192),
        ],
        out_specs=pl.BlockSpec((T, R_GRU, 64), lambda r: (0, r, 0)),
        out_shape=jax.ShapeDtypeStruct((T, NB, 64), F32),
    )(s_tm, Wih0Tb, bih0, WhhBlkb, bhh0, bhh1, Wih1Tb, bih1)


def _attn_kernel(rnn_ref, wqt_ref, bq_ref, wkt_ref, bk_ref, wvt_ref, bv_ref,
                 wot_ref, bo_ref, lng_ref, lnb_ref, wlt_ref, bl_ref,
                 wpt_ref, out_ref):
    R = R_ATT
    rnn3 = rnn_ref[...]
    rnn2 = rnn3.reshape(R * T, 64)
    rb = rnn2.astype(BF16)
    q2 = _mm(rb, wqt_ref[...]) + bq_ref[...]
    k2 = _mm(rb, wkt_ref[...]) + bk_ref[...]
    v2 = _mm(rb, wvt_ref[...]) + bv_ref[...]

    def heads(p2):
        parts = [p2[:, h * DK:(h + 1) * DK].reshape(R, T, DK)
                 for h in range(NH)]
        return jnp.concatenate(parts, axis=1).astype(BF16)

    qc = heads(q2)
    kc = heads(k2)
    vc = heads(v2)
    s_full = jax.lax.dot_general(
        qc, kc, (((2,), (2,)), ((0,), (0,))),
        preferred_element_type=F32) * (1.0 / math.sqrt(DK))
    hi = jax.lax.broadcasted_iota(jnp.int32, (NH * T, NH * T), 0)
    hj = jax.lax.broadcasted_iota(jnp.int32, (NH * T, NH * T), 1)
    allowed = ((hi // T) == (hj // T)) & ((hj % T) <= (hi % T))
    s_full = jnp.where(allowed[None], s_full, -1e9)
    m = jnp.max(s_full, axis=-1, keepdims=True)
    p = jnp.exp(s_full - m)
    p = (p / jnp.sum(p, axis=-1, keepdims=True)).astype(BF16)
    o_full = jax.lax.dot_general(
        p, vc, (((2,), (1,)), ((0,), (0,))), preferred_element_type=F32)
    o2 = jnp.concatenate(
        [o_full[:, h * T:(h + 1) * T, :].reshape(R * T, DV)
         for h in range(NH)], axis=1)
    out = _mm(o2, wot_ref[...]) + bo_ref[...] + rnn2
    mu = jnp.mean(out, axis=-1, keepdims=True)
    var = jnp.mean((out - mu) ** 2, axis=-1, keepdims=True)
    out = (out - mu) * jax.lax.rsqrt(var + 1e-6) * lng_ref[...] + lnb_ref[...]
    out3 = out.reshape(R, T, 64)
    hg = bl_ref[...]
    for t in range(T):
        hg = hg + _mm(out3[:, t, :], wlt_ref[t * 64:(t + 1) * 64, :])
    out_ref[...] = _mm(hg, wpt_ref[...]) * (64 ** -0.5)


def _attn_call(rnn_nm, WqTb, bq2, WkTb, bk2, WvTb, bv2, WoTb, bo2,
               lng2, lnb2, WlTb, bl2, WpTb):
    n_prog = NB // R_ATT
    c = lambda r, cc: pl.BlockSpec((r, cc), lambda i: (0, 0))
    return pl.pallas_call(
        _attn_kernel,
        grid=(n_prog,),
        in_specs=[
            pl.BlockSpec((R_ATT, T, 64), lambda i: (i, 0, 0)),
            c(64, 64), c(1, 64), c(64, 64), c(1, 64),
            c(64, 64), c(1, 64), c(64, 64), c(1, 64),
            c(1, 64), c(1, 64),
            c(T * 64, NHID), c(1, NHID), c(NHID, 2),
        ],
        out_specs=pl.BlockSpec((R_ATT, 2), lambda i: (i, 0)),
        out_shape=jax.ShapeDtypeStruct((NB, 2), F32),
    )(rnn_nm, WqTb, bq2, WkTb, bk2, WvTb, bv2, WoTb, bo2, lng2, lnb2,
      WlTb, bl2, WpTb)


def kernel(src_seq, H, adj, W1, b1, W2, b2, Wm, bm, Wlin, blin,
           Wih0, Whh0, bih0, bhh0, Wih1, Whh1, bih1, bhh1,
           Wq, bq, Wk, bk, Wv, bv, Wo, bo, ln_g, ln_b, Wl, bl, Wp, n_hid):
    del n_hid
    bf = lambda w: w.astype(BF16)
    hn_bf = _hn_call(H, bf(Wm.T), bm[None, :])

    xr = bf(src_seq.reshape(B, S, T * F))
    b2t = 2.0 * jnp.tile(b2, (T // 2,))[None, :]
    y_tm = _gcn_call(xr, bf(adj), hn_bf, bf(W1), b1[None, :], bf(W2), b2t,
                     bf(Wlin.T), blin[None, :])
    s_tm = y_tm.reshape(T, NB, 64)

    whhblk = jnp.zeros((128, 384), F32)
    whhblk = whhblk.at[:64, :192].set(Whh0.T)
    whhblk = whhblk.at[64:, 192:].set(Whh1.T)
    rnn_tm = _gru_call(s_tm, bf(Wih0.T), bih0[None, :], bf(whhblk),
                       bhh0[None, :], bhh1[None, :], bf(Wih1.T),
                       bih1[None, :])
    rnn_nm = jnp.transpose(rnn_tm, (1, 0, 2))

    logits = _attn_call(rnn_nm, bf(Wq.T), bq[None, :], bf(Wk.T), bk[None, :],
                        bf(Wv.T), bv[None, :], bf(Wo.T), bo[None, :],
                        ln_g[None, :], ln_b[None, :],
                        bf(Wl.T), bl[None, :], bf(Wp.T))
    return logits


# speedup vs baseline: 1.4934x; 1.1913x over previous
import math

import jax
import jax.numpy as jnp
from jax.experimental import pallas as pl
from jax.experimental.pallas import tpu as pltpu

B, S, T, F = 8, 1024, 16, 64
NB = B * S
RNN = 64
NHID = 256
NH, DK, DV = 4, 16, 16
R_GRU = 512
R_ATT = 256
F32 = jnp.float32
BF16 = jnp.bfloat16


def _mm(a, b):
    return jax.lax.dot_general(a.astype(BF16), b, (((a.ndim - 1,), (0,)),
                                                   ((), ())),
                               preferred_element_type=F32)


def _hn_kernel(h_ref, wmt_ref, bm_ref, out_ref):
    out_ref[...] = (_mm(h_ref[...], wmt_ref[...]) + bm_ref[...]).astype(BF16)


def _hn_call(H, WmT, bm2):
    return pl.pallas_call(
        _hn_kernel,
        out_shape=jax.ShapeDtypeStruct((S, S), BF16),
    )(H, WmT, bm2)


def _gcn_kernel(x_ref, adj_ref, hn_ref, w1_ref, b1_ref, w2_ref, b2t_ref,
                wlt_ref, blin_ref, out_ref):
    xb = x_ref[0].astype(BF16)
    adj = adj_ref[...]
    hn = hn_ref[...]
    w1 = w1_ref[...]
    w2 = w2_ref[...]
    b1 = b1_ref[...]
    t1cat = jnp.concatenate(
        [_mm(xb[:, tt * 64:(tt + 1) * 64], w1).astype(BF16)
         for tt in range(8)], axis=1)
    ax1 = _mm(adj, t1cat)
    ax2 = _mm(hn, t1cat)
    s1_parts = []
    s2_parts = []
    for tt in range(8):
        sl = slice(tt * 256, (tt + 1) * 256)
        o1 = jax.nn.relu(ax1[:, sl] + b1)
        o2 = jax.nn.relu(ax2[:, sl] + b1)
        s1_parts.append(_mm(o1, w2).astype(BF16))
        s2_parts.append(_mm(o2, w2).astype(BF16))
    s1 = jnp.concatenate(s1_parts, axis=1)
    s2 = jnp.concatenate(s2_parts, axis=1)
    y = _mm(adj, s1) + _mm(hn, s2) + b2t_ref[...]
    wlt = wlt_ref[...]
    blin = blin_ref[...]
    for tt in range(8):
        sl = slice(tt * 64, (tt + 1) * 64)
        out_ref[tt, 0] = (_mm(y[:, sl], wlt) + blin).astype(BF16)


def _gcn_call(xr, adj_bf, hn_bf, W1b, b12, W2b, b2t, WlinTb, blin2):
    cb = lambda r, cc: pl.BlockSpec((r, cc), lambda b, h: (0, 0))
    return pl.pallas_call(
        _gcn_kernel,
        grid=(B, 2),
        in_specs=[
            pl.BlockSpec((1, S, 512), lambda b, h: (b, 0, h)),
            cb(S, S), cb(S, S),
            cb(64, NHID), cb(1, NHID),
            cb(NHID, 64), cb(1, 512),
            cb(64, 64), cb(1, 64),
        ],
        out_specs=pl.BlockSpec((8, 1, S, 64), lambda b, h: (h, b, 0, 0)),
        out_shape=jax.ShapeDtypeStruct((T, B, S, 64), BF16),
    )(xr, adj_bf, hn_bf, W1b, b12, W2b, b2t, WlinTb, blin2)


def _gru_kernel(x_ref, wih0t_ref, bih0_ref, whh0t_ref, whh1t_ref,
                bhh0_ref, bhh1_ref, wih1t_ref, bih1_ref, out_ref):
    x3 = x_ref[...]
    x2 = x3.reshape(T * R_GRU, 64)
    gx0 = _mm(x2, wih0t_ref[...]) + bih0_ref[...]
    gx0 = gx0.reshape(T, R_GRU, 3 * RNN)
    whh0t = whh0t_ref[...]
    whh1t = whh1t_ref[...]
    wih1t = wih1t_ref[...]
    bhh0 = bhh0_ref[...]
    bhh1 = bhh1_ref[...]
    bih1 = bih1_ref[...]
    h0 = jnp.zeros((R_GRU, RNN), F32)
    h1 = jnp.zeros((R_GRU, RNN), F32)
    for t in range(T):
        gh0 = _mm(h0, whh0t) + bhh0
        gh1 = _mm(h1, whh1t) + bhh1
        g0 = gx0[t]
        rz0 = jax.nn.sigmoid(g0[:, :128] + gh0[:, :128])
        n0 = jnp.tanh(g0[:, 128:] + rz0[:, :64] * gh0[:, 128:])
        z0 = rz0[:, 64:128]
        h0 = n0 + z0 * (h0 - n0)
        g1 = _mm(h0, wih1t) + bih1
        rz1 = jax.nn.sigmoid(g1[:, :128] + gh1[:, :128])
        n1 = jnp.tanh(g1[:, 128:] + rz1[:, :64] * gh1[:, 128:])
        z1 = rz1[:, 64:128]
        h1 = n1 + z1 * (h1 - n1)
        out_ref[t] = h1


def _gru_call(s_tm, Wih0Tb, bih0, Whh0Tb, Whh1Tb, bhh0, bhh1, Wih1Tb, bih1):
    n_prog = NB // R_GRU
    c = lambda r, cc: pl.BlockSpec((r, cc), lambda i: (0, 0))
    return pl.pallas_call(
        _gru_kernel,
        grid=(n_prog,),
        in_specs=[
            pl.BlockSpec((T, R_GRU, 64), lambda r: (0, r, 0)),
            c(64, 192), c(1, 192),
            c(64, 192), c(64, 192), c(1, 192), c(1, 192),
            c(64, 192), c(1, 192),
        ],
        out_specs=pl.BlockSpec((T, R_GRU, 64), lambda r: (0, r, 0)),
        out_shape=jax.ShapeDtypeStruct((T, NB, 64), F32),
    )(s_tm, Wih0Tb, bih0, Whh0Tb, Whh1Tb, bhh0, bhh1, Wih1Tb, bih1)


def _attn_kernel(rnn_ref, wqt_ref, bq_ref, wkt_ref, bk_ref, wvt_ref, bv_ref,
                 wot_ref, bo_ref, lng_ref, lnb_ref, wlt_ref, bl_ref,
                 wpt_ref, mask_ref, seg_ref, out_ref):
    R = R_ATT
    rnn3 = rnn_ref[...]
    rnn2 = rnn3.reshape(R * T, 64)
    rb = rnn2.astype(BF16)
    q2 = (_mm(rb, wqt_ref[...]) + bq_ref[...]) * (1.0 / math.sqrt(DK))
    k2 = _mm(rb, wkt_ref[...]) + bk_ref[...]
    v2 = _mm(rb, wvt_ref[...]) + bv_ref[...]

    def hsplit(p2):
        p2b = p2.astype(BF16)
        return [p2b[:, h * DK:(h + 1) * DK].reshape(R, T, DK)
                for h in range(NH)]

    qh = hsplit(q2)
    kh = hsplit(k2)
    vh = hsplit(v2)
    s3 = jnp.concatenate(
        [jax.lax.dot_general(qh[h], kh[h], (((2,), (2,)), ((0,), (0,))),
                             preferred_element_type=F32)
         for h in range(NH)], axis=2)
    s3 = s3 + mask_ref[...]
    m = jnp.max(s3, axis=-1, keepdims=True)
    e = jnp.exp(s3 - m).reshape(R * T, NH * T)
    z = _mm(e, seg_ref[...])
    p = (e / z).astype(BF16)
    o2 = jnp.concatenate(
        [jax.lax.dot_general(p[:, h * T:(h + 1) * T].reshape(R, T, T), vh[h],
                             (((2,), (1,)), ((0,), (0,))),
                             preferred_element_type=F32).reshape(R * T, DV)
         for h in range(NH)], axis=1)
    out = _mm(o2, wot_ref[...]) + bo_ref[...] + rnn2
    mu = jnp.mean(out, axis=-1, keepdims=True)
    var = jnp.mean((out - mu) ** 2, axis=-1, keepdims=True)
    out = (out - mu) * jax.lax.rsqrt(var + 1e-6) * lng_ref[...] + lnb_ref[...]
    out3 = out.reshape(R, T, 64)
    hg = bl_ref[...]
    for t in range(T):
        hg = hg + _mm(out3[:, t, :], wlt_ref[t * 64:(t + 1) * 64, :])
    out_ref[...] = _mm(hg, wpt_ref[...]) * (64 ** -0.5)


def _attn_call(rnn_nm, WqTb, bq2, WkTb, bk2, WvTb, bv2, WoTb, bo2,
               lng2, lnb2, WlTb, bl2, WpTb, mask, seg):
    n_prog = NB // R_ATT
    c = lambda r, cc: pl.BlockSpec((r, cc), lambda i: (0, 0))
    return pl.pallas_call(
        _attn_kernel,
        grid=(n_prog,),
        in_specs=[
            pl.BlockSpec((R_ATT, T, 64), lambda i: (i, 0, 0)),
            c(64, 64), c(1, 64), c(64, 64), c(1, 64),
            c(64, 64), c(1, 64), c(64, 64), c(1, 64),
            c(1, 64), c(1, 64),
            c(T * 64, NHID), c(1, NHID), c(NHID, 2),
            c(T, NH * T), c(NH * T, NH * T),
        ],
        out_specs=pl.BlockSpec((R_ATT, 2), lambda i: (i, 0)),
        out_shape=jax.ShapeDtypeStruct((NB, 2), F32),
    )(rnn_nm, WqTb, bq2, WkTb, bk2, WvTb, bv2, WoTb, bo2, lng2, lnb2,
      WlTb, bl2, WpTb, mask, seg)


def kernel(src_seq, H, adj, W1, b1, W2, b2, Wm, bm, Wlin, blin,
           Wih0, Whh0, bih0, bhh0, Wih1, Whh1, bih1, bhh1,
           Wq, bq, Wk, bk, Wv, bv, Wo, bo, ln_g, ln_b, Wl, bl, Wp, n_hid):
    del n_hid
    bf = lambda w: w.astype(BF16)
    hn_bf = _hn_call(H, bf(Wm.T), bm[None, :])

    xr = src_seq.reshape(B, S, T * F)
    b2t = 2.0 * jnp.tile(b2, (T // 2,))[None, :]
    y_tm = _gcn_call(xr, bf(adj), hn_bf, bf(W1), b1[None, :], bf(W2), b2t,
                     bf(Wlin.T), blin[None, :])
    s_tm = y_tm.reshape(T, NB, 64)

    rnn_tm = _gru_call(s_tm, bf(Wih0.T), bih0[None, :], bf(Whh0.T),
                       bf(Whh1.T), bhh0[None, :], bhh1[None, :], bf(Wih1.T),
                       bih1[None, :])
    rnn_nm = jnp.transpose(rnn_tm, (1, 0, 2))

    ii = jnp.arange(T)
    jj = jnp.arange(NH * T) % T
    mask = jnp.where(jj[None, :] <= ii[:, None], 0.0, -1e9).astype(F32)
    seg = ((jnp.arange(NH * T)[:, None] // T)
           == (jnp.arange(NH * T)[None, :] // T)).astype(BF16)
    logits = _attn_call(rnn_nm, bf(Wq.T), bq[None, :], bf(Wk.T), bk[None, :],
                        bf(Wv.T), bv[None, :], bf(Wo.T), bo[None, :],
                        ln_g[None, :], ln_b[None, :],
                        bf(Wl.T), bl[None, :], bf(Wp.T), mask, seg)
    return logits
